# baseline (speedup 1.0000x reference)
import functools

import jax
import jax.numpy as jnp
from jax.experimental import pallas as pl
from jax.experimental.pallas import tpu as pltpu

R = 2048
C = 128
NTOT = R * C
LOGN = 18
NREAL = 384 * 384
PADROWS = (NTOT - NREAL) // C
NSAMP = 16


def _shift_down(x, s):
    return jnp.concatenate([jnp.zeros((s, x.shape[1]), x.dtype), x[:-s]], axis=0)


def _roll_axis(x, s, axis):
    if axis == 0:
        return jnp.concatenate([x[-s:, :], x[:-s, :]], axis=0)
    return jnp.concatenate([x[:, -s:], x[:, :-s]], axis=1)


def _lovasz_body(pred_ref, lab_ref, out_ref, key_ref, labv_ref):
    step = pl.program_id(0)

    logits = pred_ref[...].reshape(NREAL // C, C)
    labf = lab_ref[...].reshape(NREAL // C, C)

    signs = 2.0 * labf - 1.0
    errors = 1.0 - logits * signs

    key_ref[...] = jnp.concatenate(
        [-errors, jnp.full((PADROWS, C), jnp.inf, jnp.float32)], axis=0)
    labv_ref[...] = jnp.concatenate(
        [labf, jnp.zeros((PADROWS, C), jnp.float32)], axis=0)

    row_iota = jax.lax.broadcasted_iota(jnp.int32, (R, 1), 0)
    lane_iota = jax.lax.broadcasted_iota(jnp.int32, (1, C), 1)

    def bit_of(idx_bit):
        if idx_bit < 11:
            return (row_iota >> idx_bit) & 1
        return (lane_iota >> (idx_bit - 11)) & 1

    def exchange(pk, plb, bitj, bitk):
        key = key_ref[...]
        lab = labv_ref[...]
        keep_min = bitj == bitk
        take = (keep_min & (pk < key)) | (~keep_min & (pk > key))
        key_ref[...] = jnp.where(take, pk, key)
        labv_ref[...] = jnp.where(take, plb, lab)

    def row_pass(j, bitk):
        s = jnp.int32(1) << j
        key = key_ref[...]
        lab = labv_ref[...]
        bitj = (row_iota >> j) & 1
        is_upper = bitj == 1
        s_neg = jnp.int32(R) - s
        pk = jnp.where(is_upper, pltpu.roll(key, s, 0),
                       pltpu.roll(key, s_neg, 0))
        plb = jnp.where(is_upper, pltpu.roll(lab, s, 0),
                        pltpu.roll(lab, s_neg, 0))
        exchange(pk, plb, bitj, bitk)

    def lane_pass(j, bitk):
        s = 1 << (j - 11)
        key = key_ref[...]
        lab = labv_ref[...]
        bitj = (lane_iota >> (j - 11)) & 1
        is_upper = bitj == 1
        pk = jnp.where(is_upper, _roll_axis(key, s, 1), _roll_axis(key, -s, 1))
        plb = jnp.where(is_upper, _roll_axis(lab, s, 1), _roll_axis(lab, -s, 1))
        exchange(pk, plb, bitj, bitk)

    for k in range(1, LOGN + 1):
        if k < 11:
            bitk = (row_iota >> k) & 1
        elif k < LOGN:
            bitk = (lane_iota >> (k - 11)) & 1
        else:
            bitk = jnp.zeros((1, 1), jnp.int32)
        for j in range(k - 1, 10, -1):
            lane_pass(j, bitk)
        j_hi = min(k - 1, 10)
        jax.lax.fori_loop(
            0, j_hi + 1,
            lambda t, _, j_hi=j_hi, bitk=bitk: (row_pass(j_hi - t, bitk), 0)[1],
            0)

    key = key_ref[...]
    lab = labv_ref[...]
    e_sorted = -key

    csum = lab
    s = 1
    while s < R:
        csum = csum + _shift_down(csum, s)
        s *= 2
    tot = csum[R - 1:R, :]
    inc = tot
    s = 1
    while s < C:
        inc = inc + jnp.concatenate(
            [jnp.zeros((1, s), jnp.float32), inc[:, :-s]], axis=1)
        s *= 2
    offs = inc - tot
    kcum = csum + offs

    p_total = inc[0, C - 1]

    pos = (row_iota + R * lane_iota).astype(jnp.float32)
    inter = p_total - kcum
    union = p_total + (pos + 1.0) - kcum
    jac = 1.0 - inter / (union + 1e-07)

    carry = jnp.concatenate(
        [jnp.zeros((1, 1), jnp.float32), jac[R - 1:R, :-1]], axis=1)
    jac_m1 = jnp.concatenate([carry, jac[:-1, :]], axis=0)

    loss = jnp.sum(jnp.maximum(e_sorted, 0.0) * (jac - jac_m1)) / NSAMP

    @pl.when(step == 0)
    def _():
        out_ref[...] = jnp.zeros((1, C), jnp.float32)

    out_ref[...] += jnp.full((1, C), loss, jnp.float32)


@jax.jit
def kernel(pred, target):
    labf = target.astype(jnp.float32)
    out = pl.pallas_call(
        _lovasz_body,
        grid=(NSAMP,),
        in_specs=[
            pl.BlockSpec((1, 384, 384), lambda i: (i, 0, 0)),
            pl.BlockSpec((1, 384, 384), lambda i: (i, 0, 0)),
        ],
        out_specs=pl.BlockSpec((1, C), lambda i: (0, 0)),
        out_shape=jax.ShapeDtypeStruct((1, C), jnp.float32),
        scratch_shapes=[
            pltpu.VMEM((R, C), jnp.float32),
            pltpu.VMEM((R, C), jnp.float32),
        ],
    )(pred, labf)
    return out[0, 0]


# speedup vs baseline: 1.7785x; 1.7785x over previous
import jax
import jax.numpy as jnp
from jax.experimental import pallas as pl
from jax.experimental.pallas import tpu as pltpu

R = 2048
C = 128
NTOT = R * C
LOGN = 18
NREAL = 384 * 384
RREAL = NREAL // C
PADROWS = R - RREAL
NSAMP = 16
SB = 4
RS = SB * R


def _roll_lanes(x, s):
    return jnp.concatenate([x[:, -s:], x[:, :-s]], axis=1)


def _shift_down(x, s):
    return jnp.concatenate([jnp.zeros((s, x.shape[1]), x.dtype), x[:-s]], axis=0)


def _lovasz_body(pred_ref, lab_ref, out_ref, key_ref):
    step = pl.program_id(0)

    logits = pred_ref[...].reshape(SB, RREAL, C)
    labi = lab_ref[...].reshape(SB, RREAL, C)

    labf = labi.astype(jnp.float32)
    signs = 2.0 * labf - 1.0
    x = logits * signs - 1.0
    b = pltpu.bitcast(x, jnp.int32)
    mk = jnp.where(b >= 0, b, b ^ jnp.int32(0x7FFFFFFF))
    packed = (mk & jnp.int32(~1)) | labi

    pad_key = jnp.int32(0x7F800000)
    blocks = []
    pad = jnp.full((PADROWS, C), pad_key, jnp.int32)
    for sidx in range(SB):
        blocks.append(packed[sidx])
        blocks.append(pad)
    key_ref[...] = jnp.concatenate(blocks, axis=0)

    row_iota = jax.lax.broadcasted_iota(jnp.int32, (RS, 1), 0)
    lane_iota = jax.lax.broadcasted_iota(jnp.int32, (1, C), 1)

    def exchange(pk, bitj, bitk):
        key = key_ref[...]
        keep_min = bitj == bitk
        take = (keep_min & (pk < key)) | (~keep_min & (pk > key))
        key_ref[...] = jnp.where(take, pk, key)

    def row_pass(j, bitk):
        s = jnp.int32(1) << j
        key = key_ref[...]
        bitj = (row_iota >> j) & 1
        is_upper = bitj == 1
        pk = jnp.where(is_upper, pltpu.roll(key, s, 0),
                       pltpu.roll(key, jnp.int32(RS) - s, 0))
        exchange(pk, bitj, bitk)

    def lane_pass(j, bitk):
        s = 1 << (j - 11)
        key = key_ref[...]
        bitj = (lane_iota >> (j - 11)) & 1
        is_upper = bitj == 1
        pk = jnp.where(is_upper, _roll_lanes(key, s), _roll_lanes(key, -s))
        exchange(pk, bitj, bitk)

    for k in range(1, LOGN + 1):
        if k < 11:
            bitk = (row_iota >> k) & 1
        elif k < LOGN:
            bitk = (lane_iota >> (k - 11)) & 1
        else:
            bitk = jnp.zeros((1, 1), jnp.int32)
        for j in range(k - 1, 10, -1):
            lane_pass(j, bitk)
        j_hi = min(k - 1, 10)
        jax.lax.fori_loop(
            0, j_hi + 1,
            lambda t, _, j_hi=j_hi, bitk=bitk: (row_pass(j_hi - t, bitk), 0)[1],
            0)

    row1 = jax.lax.broadcasted_iota(jnp.int32, (R, 1), 0)
    lane1 = jax.lax.broadcasted_iota(jnp.int32, (1, C), 1)
    pos = (row1 + R * lane1).astype(jnp.float32)

    loss_sum = jnp.float32(0.0)
    for sidx in range(SB):
        kp = key_ref[sidx * R:(sidx + 1) * R, :]
        lab = (kp & 1).astype(jnp.float32)
        bdec = jnp.where(kp >= 0, kp, kp ^ jnp.int32(0x7FFFFFFF))
        e_sorted = -pltpu.bitcast(bdec, jnp.float32)

        csum = lab
        s = 1
        while s < R:
            csum = csum + _shift_down(csum, s)
            s *= 2
        tot = csum[R - 1:R, :]
        inc = tot
        s = 1
        while s < C:
            inc = inc + jnp.concatenate(
                [jnp.zeros((1, s), jnp.float32), inc[:, :-s]], axis=1)
            s *= 2
        kcum = csum + (inc - tot)
        p_total = inc[0, C - 1]

        inter = p_total - kcum
        union = p_total + (pos + 1.0) - kcum
        jac = 1.0 - inter / (union + 1e-07)

        carry = jnp.concatenate(
            [jnp.zeros((1, 1), jnp.float32), jac[R - 1:R, :-1]], axis=1)
        jac_m1 = jnp.concatenate([carry, jac[:-1, :]], axis=0)

        loss_sum += jnp.sum(jnp.maximum(e_sorted, 0.0) * (jac - jac_m1))

    @pl.when(step == 0)
    def _():
        out_ref[...] = jnp.zeros((1, C), jnp.float32)

    out_ref[...] += jnp.full((1, C), loss_sum / NSAMP, jnp.float32)


@jax.jit
def kernel(pred, target):
    labi = target.astype(jnp.int32)
    out = pl.pallas_call(
        _lovasz_body,
        grid=(NSAMP // SB,),
        in_specs=[
            pl.BlockSpec((SB, 384, 384), lambda i: (i, 0, 0)),
            pl.BlockSpec((SB, 384, 384), lambda i: (i, 0, 0)),
        ],
        out_specs=pl.BlockSpec((1, C), lambda i: (0, 0)),
        out_shape=jax.ShapeDtypeStruct((1, C), jnp.float32),
        scratch_shapes=[
            pltpu.VMEM((RS, C), jnp.int32),
        ],
    )(pred, labi)
    return out[0, 0]


# speedup vs baseline: 8.5602x; 4.8133x over previous
import jax
import jax.numpy as jnp
from jax.experimental import pallas as pl
from jax.experimental.pallas import tpu as pltpu

R = 2048
C = 128
NTOT = R * C
LOGN = 18
NREAL = 384 * 384
RREAL = NREAL // C
PADROWS = R - RREAL
NSAMP = 16
SB = 4
RS = SB * R


def _roll_lanes(x, s):
    return jnp.concatenate([x[:, -s:], x[:, :-s]], axis=1)


def _shift_down(x, s):
    return jnp.concatenate([jnp.zeros((s, x.shape[1]), x.dtype), x[:-s]], axis=0)


def _lovasz_body(pred_ref, lab_ref, out_ref, key_ref):
    step = pl.program_id(0)

    logits = pred_ref[...].reshape(SB, RREAL, C)
    labi = lab_ref[...].reshape(SB, RREAL, C)

    labf = labi.astype(jnp.float32)
    signs = 2.0 * labf - 1.0
    x = logits * signs - 1.0
    b = pltpu.bitcast(x, jnp.int32)
    mk = jnp.where(b >= 0, b, b ^ jnp.int32(0x7FFFFFFF))
    packed = (mk & jnp.int32(~1)) | labi

    pad_key = jnp.int32(0x7F800000)
    blocks = []
    pad = jnp.full((PADROWS, C), pad_key, jnp.int32)
    for sidx in range(SB):
        blocks.append(packed[sidx])
        blocks.append(pad)
    key_ref[...] = jnp.concatenate(blocks, axis=0)

    row_iota = jax.lax.broadcasted_iota(jnp.int32, (RS, 1), 0)
    lane_iota = jax.lax.broadcasted_iota(jnp.int32, (1, C), 1)

    def exchange(pk, bitj, bitk):
        key = key_ref[...]
        keep_min = bitj == bitk
        take = (keep_min & (pk < key)) | (~keep_min & (pk > key))
        key_ref[...] = jnp.where(take, pk, key)

    def row_pass_small(j, k):
        s = 1 << j
        key = key_ref[...]
        bitj = (row_iota >> j) & 1
        is_upper = bitj == 1
        pk = jnp.where(is_upper, pltpu.roll(key, s, 0), pltpu.roll(key, RS - s, 0))
        bitk = _bitk_mask(k)
        exchange(pk, bitj, bitk)

    def row_pass_halves(j, k):
        s = 1 << j
        g = RS // (2 * s)
        v = key_ref[...].reshape(g, 2, s, C)
        a = v[:, 0]
        b = v[:, 1]
        if k < 11:
            asc = ((jax.lax.broadcasted_iota(jnp.int32, (g, 1, 1), 0)
                    >> (k - j - 1)) & 1) == 0
        elif k < LOGN:
            asc = ((jax.lax.broadcasted_iota(jnp.int32, (1, 1, C), 2)
                    >> (k - 11)) & 1) == 0
        else:
            asc = jnp.ones((1, 1, 1), jnp.bool_)
        mn = jnp.minimum(a, b)
        mx = jnp.maximum(a, b)
        na = jnp.where(asc, mn, mx)
        nb = jnp.where(asc, mx, mn)
        key_ref[...] = jnp.concatenate(
            [na[:, None], nb[:, None]], axis=1).reshape(RS, C)

    def _bitk_mask(k):
        if k < 11:
            return (row_iota >> k) & 1
        if k < LOGN:
            return (lane_iota >> (k - 11)) & 1
        return jnp.zeros((1, 1), jnp.int32)

    def lane_pass(j, k):
        s = 1 << (j - 11)
        key = key_ref[...]
        bitj = (lane_iota >> (j - 11)) & 1
        is_upper = bitj == 1
        pk = jnp.where(is_upper, _roll_lanes(key, s), _roll_lanes(key, -s))
        exchange(pk, bitj, _bitk_mask(k))

    for k in range(1, LOGN + 1):
        for j in range(k - 1, -1, -1):
            if j >= 11:
                lane_pass(j, k)
            elif j >= 3:
                row_pass_halves(j, k)
            else:
                row_pass_small(j, k)

    row1 = jax.lax.broadcasted_iota(jnp.int32, (R, 1), 0)
    lane1 = jax.lax.broadcasted_iota(jnp.int32, (1, C), 1)
    pos = (row1 + R * lane1).astype(jnp.float32)

    loss_sum = jnp.float32(0.0)
    for sidx in range(SB):
        kp = key_ref[sidx * R:(sidx + 1) * R, :]
        lab = (kp & 1).astype(jnp.float32)
        bdec = jnp.where(kp >= 0, kp, kp ^ jnp.int32(0x7FFFFFFF))
        e_sorted = -pltpu.bitcast(bdec, jnp.float32)

        csum = lab
        s = 1
        while s < R:
            csum = csum + _shift_down(csum, s)
            s *= 2
        tot = csum[R - 1:R, :]
        inc = tot
        s = 1
        while s < C:
            inc = inc + jnp.concatenate(
                [jnp.zeros((1, s), jnp.float32), inc[:, :-s]], axis=1)
            s *= 2
        kcum = csum + (inc - tot)
        p_total = inc[0, C - 1]

        inter = p_total - kcum
        union = p_total + (pos + 1.0) - kcum
        jac = 1.0 - inter / (union + 1e-07)

        carry = jnp.concatenate(
            [jnp.zeros((1, 1), jnp.float32), jac[R - 1:R, :-1]], axis=1)
        jac_m1 = jnp.concatenate([carry, jac[:-1, :]], axis=0)

        loss_sum += jnp.sum(jnp.maximum(e_sorted, 0.0) * (jac - jac_m1))

    @pl.when(step == 0)
    def _():
        out_ref[...] = jnp.zeros((1, C), jnp.float32)

    out_ref[...] += jnp.full((1, C), loss_sum / NSAMP, jnp.float32)


@jax.jit
def kernel(pred, target):
    labi = target.astype(jnp.int32)
    out = pl.pallas_call(
        _lovasz_body,
        grid=(NSAMP // SB,),
        in_specs=[
            pl.BlockSpec((SB, 384, 384), lambda i: (i, 0, 0)),
            pl.BlockSpec((SB, 384, 384), lambda i: (i, 0, 0)),
        ],
        out_specs=pl.BlockSpec((1, C), lambda i: (0, 0)),
        out_shape=jax.ShapeDtypeStruct((1, C), jnp.float32),
        scratch_shapes=[
            pltpu.VMEM((RS, C), jnp.int32),
        ],
    )(pred, labi)
    return out[0, 0]


# speedup vs baseline: 8.7680x; 1.0243x over previous
import jax
import jax.numpy as jnp
from jax.experimental import pallas as pl
from jax.experimental.pallas import tpu as pltpu

R = 2048
C = 128
NTOT = R * C
LOGN = 18
NREAL = 384 * 384
RREAL = NREAL // C
PADROWS = R - RREAL
NSAMP = 16
SB = 8
RS = SB * R


def _roll_lanes(x, s):
    return jnp.concatenate([x[:, -s:], x[:, :-s]], axis=1)


def _shift_down(x, s):
    return jnp.concatenate([jnp.zeros((s, x.shape[1]), x.dtype), x[:-s]], axis=0)


def _lovasz_body(packed_ref, out_ref, key_ref):
    step = pl.program_id(0)

    packed = packed_ref[...].reshape(SB, RREAL, C)

    pad_key = jnp.int32(0x7F800000)
    blocks = []
    pad = jnp.full((PADROWS, C), pad_key, jnp.int32)
    for sidx in range(SB):
        blocks.append(packed[sidx])
        blocks.append(pad)
    key_ref[...] = jnp.concatenate(blocks, axis=0)

    row_iota = jax.lax.broadcasted_iota(jnp.int32, (RS, 1), 0)
    lane_iota = jax.lax.broadcasted_iota(jnp.int32, (1, C), 1)

    def exchange(pk, bitj, bitk):
        key = key_ref[...]
        keep_min = bitj == bitk
        take = (keep_min & (pk < key)) | (~keep_min & (pk > key))
        key_ref[...] = jnp.where(take, pk, key)

    def row_pass_small(j, k):
        s = 1 << j
        key = key_ref[...]
        bitj = (row_iota >> j) & 1
        is_upper = bitj == 1
        pk = jnp.where(is_upper, pltpu.roll(key, s, 0), pltpu.roll(key, RS - s, 0))
        bitk = _bitk_mask(k)
        exchange(pk, bitj, bitk)

    def row_pass_halves(j, k):
        s = 1 << j
        g = RS // (2 * s)
        v = key_ref[...].reshape(g, 2, s, C)
        a = v[:, 0]
        b = v[:, 1]
        if k < 11:
            asc = ((jax.lax.broadcasted_iota(jnp.int32, (g, 1, 1), 0)
                    >> (k - j - 1)) & 1) == 0
        elif k < LOGN:
            asc = ((jax.lax.broadcasted_iota(jnp.int32, (1, 1, C), 2)
                    >> (k - 11)) & 1) == 0
        else:
            asc = jnp.ones((1, 1, 1), jnp.bool_)
        mn = jnp.minimum(a, b)
        mx = jnp.maximum(a, b)
        na = jnp.where(asc, mn, mx)
        nb = jnp.where(asc, mx, mn)
        key_ref[...] = jnp.concatenate(
            [na[:, None], nb[:, None]], axis=1).reshape(RS, C)

    def _bitk_mask(k):
        if k < 11:
            return (row_iota >> k) & 1
        if k < LOGN:
            return (lane_iota >> (k - 11)) & 1
        return jnp.zeros((1, 1), jnp.int32)

    def lane_pass(j, k):
        s = 1 << (j - 11)
        key = key_ref[...]
        bitj = (lane_iota >> (j - 11)) & 1
        is_upper = bitj == 1
        pk = jnp.where(is_upper, _roll_lanes(key, s), _roll_lanes(key, -s))
        exchange(pk, bitj, _bitk_mask(k))

    for k in range(1, LOGN + 1):
        for j in range(k - 1, -1, -1):
            if j >= 11:
                lane_pass(j, k)
            elif j >= 3:
                row_pass_halves(j, k)
            else:
                row_pass_small(j, k)

    row1 = jax.lax.broadcasted_iota(jnp.int32, (R, 1), 0)
    lane1 = jax.lax.broadcasted_iota(jnp.int32, (1, C), 1)
    pos = (row1 + R * lane1).astype(jnp.float32)

    loss_sum = jnp.float32(0.0)
    for sidx in range(SB):
        kp = key_ref[sidx * R:(sidx + 1) * R, :]
        lab = (kp & 1).astype(jnp.float32)
        bdec = jnp.where(kp >= 0, kp, kp ^ jnp.int32(0x7FFFFFFF))
        e_sorted = -pltpu.bitcast(bdec, jnp.float32)

        csum = lab
        s = 1
        while s < R:
            csum = csum + _shift_down(csum, s)
            s *= 2
        tot = csum[R - 1:R, :]
        inc = tot
        s = 1
        while s < C:
            inc = inc + jnp.concatenate(
                [jnp.zeros((1, s), jnp.float32), inc[:, :-s]], axis=1)
            s *= 2
        kcum = csum + (inc - tot)
        p_total = inc[0, C - 1]

        inter = p_total - kcum
        union = p_total + (pos + 1.0) - kcum
        jac = 1.0 - inter / (union + 1e-07)

        carry = jnp.concatenate(
            [jnp.zeros((1, 1), jnp.float32), jac[R - 1:R, :-1]], axis=1)
        jac_m1 = jnp.concatenate([carry, jac[:-1, :]], axis=0)

        loss_sum += jnp.sum(jnp.maximum(e_sorted, 0.0) * (jac - jac_m1))

    @pl.when(step == 0)
    def _():
        out_ref[...] = jnp.zeros((1, C), jnp.float32)

    out_ref[...] += jnp.full((1, C), loss_sum / NSAMP, jnp.float32)


@jax.jit
def kernel(pred, target):
    labi = target.astype(jnp.int32)
    signs = 2.0 * labi.astype(jnp.float32) - 1.0
    x = pred * signs - 1.0
    b = jax.lax.bitcast_convert_type(x, jnp.int32)
    mk = jnp.where(b >= 0, b, b ^ jnp.int32(0x7FFFFFFF))
    packed = (mk & jnp.int32(~1)) | labi

    out = pl.pallas_call(
        _lovasz_body,
        grid=(NSAMP // SB,),
        in_specs=[
            pl.BlockSpec((SB, 384, 384), lambda i: (i, 0, 0)),
        ],
        out_specs=pl.BlockSpec((1, C), lambda i: (0, 0)),
        out_shape=jax.ShapeDtypeStruct((1, C), jnp.float32),
        scratch_shapes=[
            pltpu.VMEM((RS, C), jnp.int32),
        ],
    )(packed)
    return out[0, 0]
